# baseline (speedup 1.0000x reference)
import functools

import jax
import jax.numpy as jnp
from jax import lax
from jax.experimental import pallas as pl
from jax.experimental.pallas import tpu as pltpu
from jax.experimental.pallas import tpu_sc as plsc

N = 10000
E = 160000
R = 8
R9 = R + 1
D = 256
H = 128
RN = R * N
R9N = R9 * N

NC = 2
NS = 16
CH = 128

P_SUB = -(-E // (NS * 2 * CH)) * 2 * CH
E_PAD = NS * P_SUB
N_CH = P_SUB // CH
NP = N_CH // 2

ACC = 10112
DST_BITS = 14
ROWS_PER_TILE = ACC // NS
BN = 1000

BF = jnp.bfloat16


def _hrk_body(h_ref, w_ref, o_ref):
    res = jnp.dot(h_ref[...], w_ref[0], preferred_element_type=jnp.float32)
    o_ref[0, 0] = res[:, :H]
    o_ref[1, 0] = res[:, H:]


def _hrk(h, W9):
    return pl.pallas_call(
        _hrk_body,
        grid=(N // BN, R9),
        in_specs=[
            pl.BlockSpec((BN, D), lambda nb, r: (nb, 0)),
            pl.BlockSpec((1, D, D), lambda nb, r: (r, 0, 0)),
        ],
        out_specs=pl.BlockSpec((NC, 1, BN, H), lambda nb, r: (0, r, nb, 0)),
        out_shape=jax.ShapeDtypeStruct((NC, R9, N, H), jnp.float32),
    )(h, W9)


def _prep_body(agg_ref, hr_ref, b_ref, h_ref):
    hcat = (jnp.concatenate([agg_ref[0], agg_ref[1]], axis=1)
            + jnp.concatenate([hr_ref[0, 0], hr_ref[1, 0]], axis=1)
            + b_ref[...])
    h_ref[...] = jnp.maximum(hcat, 0.0)


def _prep(agg, HR, b2):
    return pl.pallas_call(
        _prep_body,
        grid=(N // BN,),
        in_specs=[
            pl.BlockSpec((NC, BN, H), lambda nb: (0, nb, 0)),
            pl.BlockSpec((NC, 1, BN, H), lambda nb: (0, R, nb, 0)),
            pl.BlockSpec((1, D), lambda nb: (0, 0)),
        ],
        out_specs=pl.BlockSpec((BN, D), lambda nb: (nb, 0)),
        out_shape=jax.ShapeDtypeStruct((N, D), jnp.float32),
    )(agg, HR, b2)


def _fin_body(agg_ref, hr_ref, b_ref, o_ref):
    nb = pl.program_id(0)
    hcat = (jnp.concatenate([agg_ref[0], agg_ref[1]], axis=1)
            + jnp.concatenate([hr_ref[0, 0], hr_ref[1, 0]], axis=1)
            + b_ref[...])
    hb = jnp.maximum(hcat, 0.0)
    part = jnp.sum(hb, axis=0, keepdims=True)

    @pl.when(nb == 0)
    def _():
        o_ref[...] = jnp.zeros_like(o_ref)

    o_ref[...] += part


def _fin(agg, HR, b2):
    return pl.pallas_call(
        _fin_body,
        grid=(N // BN,),
        in_specs=[
            pl.BlockSpec((NC, BN, H), lambda nb: (0, nb, 0)),
            pl.BlockSpec((NC, 1, BN, H), lambda nb: (0, R, nb, 0)),
            pl.BlockSpec((1, D), lambda nb: (0, 0)),
        ],
        out_specs=pl.BlockSpec((1, D), lambda nb: (0, 0)),
        out_shape=jax.ShapeDtypeStruct((1, D), jnp.float32),
    )(agg, HR, b2)


def _sc_scatter(hr_flat, pidx, zrows):
    mesh = plsc.VectorSubcoreMesh(core_axis_name="c", subcore_axis_name="s")

    @functools.partial(
        pl.kernel,
        mesh=mesh,
        out_type=jax.ShapeDtypeStruct((NC, ACC, H), jnp.float32),
        scratch_types=[
            pltpu.VMEM((P_SUB,), jnp.int32),
            pltpu.VMEM((CH,), jnp.int32),
            pltpu.VMEM((CH,), jnp.int32),
            pltpu.VMEM((CH,), jnp.int32),
            pltpu.VMEM((CH,), jnp.int32),
            pltpu.VMEM((2, CH, H), jnp.float32),
            pltpu.VMEM_SHARED((ACC, H), jnp.float32),
            pltpu.SemaphoreType.DMA,
            pltpu.SemaphoreType.DMA,
        ],
    )
    def k(hr_hbm, pidx_hbm, z_hbm, out_hbm, pks, gixb0, gixb1, dstb0, dstb1,
          rows, acc_s, sem0, sem1):
        cid = lax.axis_index("c")
        sid = lax.axis_index("s")
        row0 = sid * ROWS_PER_TILE
        pltpu.sync_copy(z_hbm.at[pl.ds(row0, ROWS_PER_TILE)],
                        acc_s.at[pl.ds(row0, ROWS_PER_TILE)])
        pltpu.sync_copy(pidx_hbm.at[pl.ds(sid * P_SUB, P_SUB)], pks)

        coff = cid * R9N
        gixb = (gixb0, gixb1)
        dstb = (dstb0, dstb1)
        sems = (sem0, sem1)

        def unpack(j, b):
            for i in range(CH // 16):
                p = pks[pl.ds(j * CH + i * 16, 16)]
                sl = pl.ds(i * 16, 16)
                gixb[b][sl] = (p >> DST_BITS) + coff
                dstb[b][sl] = p & (2**DST_BITS - 1)

        def g_start(b):
            pltpu.async_copy(hr_hbm.at[gixb[b]], rows.at[b], sems[b])

        def g_wait(b):
            pltpu.make_async_copy(hr_hbm.at[gixb[b]], rows.at[b],
                                  sems[b]).wait()

        def s_sync(b):
            pltpu.sync_copy(rows.at[b], acc_s.at[dstb[b]], add=True)

        plsc.subcore_barrier()

        unpack(0, 0)
        unpack(1, 1)
        g_start(0)
        g_start(1)

        def body(g, carry):
            j0 = 2 * g
            g_wait(0)
            s_sync(0)

            @pl.when(g < NP - 1)
            def _():
                unpack(j0 + 2, 0)
                g_start(0)

            g_wait(1)
            s_sync(1)

            @pl.when(g < NP - 1)
            def _():
                unpack(j0 + 3, 1)
                g_start(1)

            return carry

        lax.fori_loop(0, NP, body, 0)
        plsc.subcore_barrier()

        @pl.when(cid == 0)
        def _():
            pltpu.sync_copy(acc_s.at[pl.ds(row0, ROWS_PER_TILE)],
                            out_hbm.at[0, pl.ds(row0, ROWS_PER_TILE)])

        @pl.when(cid == 1)
        def _():
            pltpu.sync_copy(acc_s.at[pl.ds(row0, ROWS_PER_TILE)],
                            out_hbm.at[1, pl.ds(row0, ROWS_PER_TILE)])

    return k(hr_flat, pidx, zrows)


def kernel(x, edge_index, edge_type, W0, S0, b0, W1, S1, b1, W2, S2, b2,
           W3, S3, b3):
    src, dst = edge_index[0], edge_index[1]
    gidx = edge_type * N + src
    pad = E_PAD - E
    gidx_p = jnp.concatenate([gidx, jnp.zeros((pad,), jnp.int32)])
    dst_p = jnp.concatenate([dst, jnp.full((pad,), N, jnp.int32)])
    pidx = gidx_p * 2**DST_BITS + dst_p
    zrows = jnp.zeros((ACC, H), jnp.float32)

    layers = ((W0, S0, b0), (W1, S1, b1), (W2, S2, b2), (W3, S3, b3))
    h = x
    agg = HR = None
    blast = None
    for l in range(4):
        W, S, b = layers[l]
        W9 = jnp.concatenate([W, S.reshape(1, D, D)], axis=0)
        HR = _hrk(h, W9)
        agg = _sc_scatter(HR.reshape(NC * R9N, H), pidx, zrows)
        blast = b.reshape(1, D)
        if l < 3:
            h = _prep(agg, HR, blast)
    out = _fin(agg, HR, blast)
    return out.reshape(1, 1, D)
